# Initial kernel scaffold; baseline (speedup 1.0000x reference)
import jax
import jax.numpy as jnp
from jax.experimental import pallas as pl


def kernel(pointcloud, params):
    raise NotImplementedError("write your pallas kernel here")


# speedup vs baseline: 22.7185x; 22.7185x over previous
import functools

import jax
import jax.numpy as jnp
from jax.experimental import pallas as pl

_BIG = 1e30


def _dot(a, b, prec=jax.lax.Precision.DEFAULT):
    return jax.lax.dot_general(
        a, b, (((1,), (0,)), ((), ())),
        preferred_element_type=jnp.float32,
        precision=prec)


_EXACT = jax.lax.Precision.HIGHEST


def _first_min_idx(e, iota, n):
    m = jnp.min(e, axis=1)
    idx = jnp.min(jnp.where(e == m[:, None], iota, n), axis=1)
    return m, idx


def _fps_body(xyzT_ref, out_ref, *, npoint):
    x = xyzT_ref[:, 0, :]
    y = xyzT_ref[:, 1, :]
    z = xyzT_ref[:, 2, :]
    B, N = x.shape
    iota = jax.lax.broadcasted_iota(jnp.int32, (B, N), 1)

    def body(i, carry):
        dists, far = carry
        oh = (iota == far[:, None]).astype(jnp.float32)
        cx = jnp.sum(oh * x, axis=1)
        cy = jnp.sum(oh * y, axis=1)
        cz = jnp.sum(oh * z, axis=1)
        out_ref[0, pl.ds(i, 1), :] = cx[None, :]
        out_ref[1, pl.ds(i, 1), :] = cy[None, :]
        out_ref[2, pl.ds(i, 1), :] = cz[None, :]
        dx = x - cx[:, None]
        dy = y - cy[:, None]
        dz = z - cz[:, None]
        d = (dx * dx + dy * dy) + dz * dz
        dists = jnp.minimum(dists, d)
        dmax = jnp.max(dists, axis=1)
        far = jnp.min(jnp.where(dists == dmax[:, None], iota, N), axis=1)
        return dists, far

    dists0 = jnp.full((B, N), 1e10, jnp.float32)
    far0 = jnp.zeros((B,), jnp.int32)
    jax.lax.fori_loop(0, npoint, body, (dists0, far0))


def _fps(xyzT, npoint):
    B, _, N = xyzT.shape
    return pl.pallas_call(
        functools.partial(_fps_body, npoint=npoint),
        out_shape=jax.ShapeDtypeStruct((3, npoint, B), jnp.float32),
    )(xyzT)


def _sa_body(pdata_ref, xyzT_ref, c_ref,
             W1_ref, b1_ref, W2_ref, b2_ref, W3_ref, b3_ref,
             out_ref, *, r2, nsample):
    pdata = pdata_ref[0]
    px = xyzT_ref[0, 0][None, :]
    py = xyzT_ref[0, 1][None, :]
    pz = xyzT_ref[0, 2][None, :]
    c = c_ref[0]
    cx = c[:, 0:1]
    cy = c[:, 1:2]
    cz = c[:, 2:3]
    Ts = c.shape[0]
    N = pdata.shape[0]

    na = (cx * cx + cy * cy) + cz * cz
    nb = (px * px + py * py) + pz * pz
    pmat = jnp.concatenate([px, py, pz], axis=0)
    dotc = _dot(c, pmat)
    d = (na + nb) - 2.0 * dotc
    mask = d <= r2
    iota = jax.lax.broadcasted_iota(jnp.int32, (Ts, N), 1)
    cur = jnp.where(mask, iota, N)
    first = jnp.min(cur, axis=1)
    first = jnp.where(first == N, 0, first)
    maxcnt = jnp.max(jnp.sum(mask.astype(jnp.int32), axis=1))
    npass = jnp.maximum(jnp.minimum(maxcnt, nsample), 1)

    C3 = W3_ref.shape[1]
    acc0 = jnp.full((Ts, C3), -_BIG, jnp.float32)

    def pass_body(_, carry):
        acc, cur = carry
        m = jnp.min(cur, axis=1)
        idxk = jnp.where(m == N, first, m)
        oh = (iota == idxk[:, None]).astype(jnp.float32)
        g = _dot(oh, pdata, _EXACT)
        h = jnp.concatenate([g[:, 0:3] - c, g[:, 3:]], axis=1)
        h = jnp.maximum(_dot(h, W1_ref[...]) + b1_ref[...], 0.0)
        h = jnp.maximum(_dot(h, W2_ref[...]) + b2_ref[...], 0.0)
        h = jnp.maximum(_dot(h, W3_ref[...]) + b3_ref[...], 0.0)
        acc = jnp.maximum(acc, h)
        cur = jnp.where(cur == m[:, None], N, cur)
        return acc, cur

    acc, _ = jax.lax.fori_loop(0, npass, pass_body, (acc0, cur))
    out_ref[0] = acc


def _sa(pdata, xyzT, centers, mlp, radius, nsample, s_tile):
    B, N, Cin = pdata.shape
    S = centers.shape[1]
    (W1, b1), (W2, b2), (W3, b3) = mlp
    C3 = W3.shape[1]
    grid = (B, S // s_tile)
    return pl.pallas_call(
        functools.partial(_sa_body, r2=radius * radius, nsample=nsample),
        grid=grid,
        in_specs=[
            pl.BlockSpec((1, N, Cin), lambda b, s: (b, 0, 0)),
            pl.BlockSpec((1, 3, N), lambda b, s: (b, 0, 0)),
            pl.BlockSpec((1, s_tile, 3), lambda b, s: (b, s, 0)),
            pl.BlockSpec(W1.shape, lambda b, s: (0, 0)),
            pl.BlockSpec(b1.shape, lambda b, s: (0, 0)),
            pl.BlockSpec(W2.shape, lambda b, s: (0, 0)),
            pl.BlockSpec(b2.shape, lambda b, s: (0, 0)),
            pl.BlockSpec(W3.shape, lambda b, s: (0, 0)),
            pl.BlockSpec(b3.shape, lambda b, s: (0, 0)),
        ],
        out_specs=pl.BlockSpec((1, s_tile, C3), lambda b, s: (b, s, 0)),
        out_shape=jax.ShapeDtypeStruct((B, S, C3), jnp.float32),
    )(pdata, xyzT, centers, W1, b1, W2, b2, W3, b3)


def _fp_body(x1_ref, xyz2T_ref, f1_ref, f2_ref,
             Wa_ref, ba_ref, Wb_ref, bb_ref,
             *rest, with_cls):
    if with_cls:
        Wc1_ref, bc1_ref, Wc2_ref, bc2_ref, out_ref = rest
    else:
        (out_ref,) = rest
    x1 = x1_ref[0]
    ax = x1[:, 0:1]
    ay = x1[:, 1:2]
    az = x1[:, 2:3]
    bx = xyz2T_ref[0, 0][None, :]
    by = xyz2T_ref[0, 1][None, :]
    bz = xyz2T_ref[0, 2][None, :]
    T = x1.shape[0]
    N2 = xyz2T_ref.shape[2]

    na = (ax * ax + ay * ay) + az * az
    nb = (bx * bx + by * by) + bz * bz
    bmat = jnp.concatenate([bx, by, bz], axis=0)
    dotc = _dot(x1, bmat)
    d = (na + nb) - 2.0 * dotc
    iota = jax.lax.broadcasted_iota(jnp.int32, (T, N2), 1)

    e = d
    ws = []
    ids = []
    for _ in range(3):
        m, idx = _first_min_idx(e, iota, N2)
        dist = jnp.sqrt(jnp.maximum(m, 0.0))
        ws.append(1.0 / (dist + 1e-8))
        ids.append(idx)
        e = jnp.where(iota == idx[:, None], _BIG, e)
    wtot = (ws[0] + ws[1]) + ws[2]
    Wmat = ((ws[0] / wtot)[:, None] * (iota == ids[0][:, None])
            + (ws[1] / wtot)[:, None] * (iota == ids[1][:, None])
            + (ws[2] / wtot)[:, None] * (iota == ids[2][:, None])).astype(jnp.float32)

    interp = _dot(Wmat, f2_ref[0], _EXACT)
    h = jnp.concatenate([interp, f1_ref[0]], axis=1)
    h = jnp.maximum(_dot(h, Wa_ref[...]) + ba_ref[...], 0.0)
    h = jnp.maximum(_dot(h, Wb_ref[...]) + bb_ref[...], 0.0)
    if with_cls:
        h = jnp.maximum(_dot(h, Wc1_ref[...]) + bc1_ref[...], 0.0)
        h = _dot(h, Wc2_ref[...]) + bc2_ref[...]
    out_ref[0] = h


def _fp(xyz1, xyz2T, feats1, feats2, mlp, n_tile, cls=None):
    B, N1, _ = xyz1.shape
    N2 = xyz2T.shape[2]
    C1 = feats1.shape[2]
    C2 = feats2.shape[2]
    (Wa, ba), (Wb, bb) = mlp
    weights = [Wa, ba, Wb, bb]
    Cout = Wb.shape[1]
    if cls is not None:
        (Wc1, bc1), (Wc2, bc2) = cls
        weights += [Wc1, bc1, Wc2, bc2]
        Cout = Wc2.shape[1]
    grid = (B, N1 // n_tile)
    w_specs = [pl.BlockSpec(w.shape, lambda b, s: (0, 0)) for w in weights]
    return pl.pallas_call(
        functools.partial(_fp_body, with_cls=cls is not None),
        grid=grid,
        in_specs=[
            pl.BlockSpec((1, n_tile, 3), lambda b, s: (b, s, 0)),
            pl.BlockSpec((1, 3, N2), lambda b, s: (b, 0, 0)),
            pl.BlockSpec((1, n_tile, C1), lambda b, s: (b, s, 0)),
            pl.BlockSpec((1, N2, C2), lambda b, s: (b, 0, 0)),
        ] + w_specs,
        out_specs=pl.BlockSpec((1, n_tile, Cout), lambda b, s: (b, s, 0)),
        out_shape=jax.ShapeDtypeStruct((B, N1, Cout), jnp.float32),
    )(xyz1, xyz2T, feats1, feats2, *weights)


_NPOINTS = [1024, 256, 64, 16]
_RADIUS = [0.1, 0.2, 0.4, 0.8]
_NSAMPLE = [32, 32, 32, 32]
_SA_TILE = [256, 256, 64, 16]
_FP_TILE = [512, 256, 64, 16]


def kernel(pointcloud, params):
    pc = pointcloud.astype(jnp.float32)
    B, N, _ = pc.shape

    def prep(mlp):
        return [(W, b.reshape(1, -1)) for (W, b) in mlp]

    sa_params = [prep(m) for m in params["sa"]]
    fp_params = [prep(m) for m in params["fp"]]
    cls_params = prep(params["cls"])

    xyz = pc[..., 0:3]
    xyzT = jnp.transpose(xyz, (0, 2, 1))
    l_xyz = [xyz]
    l_xyzT = [xyzT]
    l_feats = [pc[..., 3:]]
    pdata = pc
    for k in range(4):
        ctrs = _fps(l_xyzT[k], _NPOINTS[k])
        centers = jnp.transpose(ctrs, (2, 1, 0))
        centersT = jnp.transpose(ctrs, (2, 0, 1))
        nf = _sa(pdata, l_xyzT[k], centers, sa_params[k],
                 _RADIUS[k], _NSAMPLE[k], _SA_TILE[k])
        l_xyz.append(centers)
        l_xyzT.append(centersT)
        l_feats.append(nf)
        pdata = jnp.concatenate([centers, nf], axis=-1)

    f = l_feats[4]
    for lvl in (3, 2, 1):
        f = _fp(l_xyz[lvl], l_xyzT[lvl + 1], l_feats[lvl], f,
                fp_params[lvl], _FP_TILE[lvl])
    out = _fp(l_xyz[0], l_xyzT[1], l_feats[0], f,
              fp_params[0], _FP_TILE[0], cls=cls_params)
    return out


# speedup vs baseline: 26.3234x; 1.1587x over previous
import functools

import jax
import jax.numpy as jnp
import numpy as np
from jax.experimental import pallas as pl
from jax.experimental.shard_map import shard_map
from jax.sharding import Mesh, PartitionSpec as P

_BIG = 1e30


def _dot(a, b, prec=jax.lax.Precision.DEFAULT):
    return jax.lax.dot_general(
        a, b, (((1,), (0,)), ((), ())),
        preferred_element_type=jnp.float32,
        precision=prec)


_EXACT = jax.lax.Precision.HIGHEST


def _first_min_idx(e, iota, n):
    m = jnp.min(e, axis=1)
    idx = jnp.min(jnp.where(e == m[:, None], iota, n), axis=1)
    return m, idx


def _fps_body(xyzT_ref, out_ref, *, npoint):
    x = xyzT_ref[:, 0, :]
    y = xyzT_ref[:, 1, :]
    z = xyzT_ref[:, 2, :]
    B, N = x.shape
    iota = jax.lax.broadcasted_iota(jnp.int32, (B, N), 1)

    def body(i, carry):
        dists, far = carry
        oh = (iota == far[:, None]).astype(jnp.float32)
        cx = jnp.sum(oh * x, axis=1)
        cy = jnp.sum(oh * y, axis=1)
        cz = jnp.sum(oh * z, axis=1)
        out_ref[0, pl.ds(i, 1), :] = cx[None, :]
        out_ref[1, pl.ds(i, 1), :] = cy[None, :]
        out_ref[2, pl.ds(i, 1), :] = cz[None, :]
        dx = x - cx[:, None]
        dy = y - cy[:, None]
        dz = z - cz[:, None]
        d = (dx * dx + dy * dy) + dz * dz
        dists = jnp.minimum(dists, d)
        dmax = jnp.max(dists, axis=1)
        far = jnp.min(jnp.where(dists == dmax[:, None], iota, N), axis=1)
        return dists, far

    dists0 = jnp.full((B, N), 1e10, jnp.float32)
    far0 = jnp.zeros((B,), jnp.int32)
    jax.lax.fori_loop(0, npoint, body, (dists0, far0))


def _fps(xyzT, npoint):
    B, _, N = xyzT.shape
    return pl.pallas_call(
        functools.partial(_fps_body, npoint=npoint),
        out_shape=jax.ShapeDtypeStruct((3, npoint, B), jnp.float32),
    )(xyzT)


def _sa_body(pdata_ref, xyzT_ref, c_ref,
             W1_ref, b1_ref, W2_ref, b2_ref, W3_ref, b3_ref,
             out_ref, *, r2, nsample):
    pdata = pdata_ref[0]
    px = xyzT_ref[0, 0][None, :]
    py = xyzT_ref[0, 1][None, :]
    pz = xyzT_ref[0, 2][None, :]
    c = c_ref[0]
    cx = c[:, 0:1]
    cy = c[:, 1:2]
    cz = c[:, 2:3]
    Ts = c.shape[0]
    N = pdata.shape[0]

    na = (cx * cx + cy * cy) + cz * cz
    nb = (px * px + py * py) + pz * pz
    pmat = jnp.concatenate([px, py, pz], axis=0)
    dotc = _dot(c, pmat)
    d = (na + nb) - 2.0 * dotc
    mask = d <= r2
    iota = jax.lax.broadcasted_iota(jnp.int32, (Ts, N), 1)
    cur = jnp.where(mask, iota, N)
    first = jnp.min(cur, axis=1)
    first = jnp.where(first == N, 0, first)
    maxcnt = jnp.max(jnp.sum(mask.astype(jnp.int32), axis=1))
    npass = jnp.maximum(jnp.minimum(maxcnt, nsample), 1)

    C3 = W3_ref.shape[1]
    acc0 = jnp.full((Ts, C3), -_BIG, jnp.float32)

    def pass_body(_, carry):
        acc, cur = carry
        m = jnp.min(cur, axis=1)
        idxk = jnp.where(m == N, first, m)
        oh = (iota == idxk[:, None]).astype(jnp.float32)
        g = _dot(oh, pdata, _EXACT)
        h = jnp.concatenate([g[:, 0:3] - c, g[:, 3:]], axis=1)
        h = jnp.maximum(_dot(h, W1_ref[...]) + b1_ref[...], 0.0)
        h = jnp.maximum(_dot(h, W2_ref[...]) + b2_ref[...], 0.0)
        h = jnp.maximum(_dot(h, W3_ref[...]) + b3_ref[...], 0.0)
        acc = jnp.maximum(acc, h)
        cur = jnp.where(cur == m[:, None], N, cur)
        return acc, cur

    acc, _ = jax.lax.fori_loop(0, npass, pass_body, (acc0, cur))
    out_ref[0] = acc


def _sa(pdata, xyzT, centers, mlp, radius, nsample, s_tile):
    B, N, Cin = pdata.shape
    S = centers.shape[1]
    (W1, b1), (W2, b2), (W3, b3) = mlp
    C3 = W3.shape[1]
    grid = (B, S // s_tile)
    return pl.pallas_call(
        functools.partial(_sa_body, r2=radius * radius, nsample=nsample),
        grid=grid,
        in_specs=[
            pl.BlockSpec((1, N, Cin), lambda b, s: (b, 0, 0)),
            pl.BlockSpec((1, 3, N), lambda b, s: (b, 0, 0)),
            pl.BlockSpec((1, s_tile, 3), lambda b, s: (b, s, 0)),
            pl.BlockSpec(W1.shape, lambda b, s: (0, 0)),
            pl.BlockSpec(b1.shape, lambda b, s: (0, 0)),
            pl.BlockSpec(W2.shape, lambda b, s: (0, 0)),
            pl.BlockSpec(b2.shape, lambda b, s: (0, 0)),
            pl.BlockSpec(W3.shape, lambda b, s: (0, 0)),
            pl.BlockSpec(b3.shape, lambda b, s: (0, 0)),
        ],
        out_specs=pl.BlockSpec((1, s_tile, C3), lambda b, s: (b, s, 0)),
        out_shape=jax.ShapeDtypeStruct((B, S, C3), jnp.float32),
    )(pdata, xyzT, centers, W1, b1, W2, b2, W3, b3)


def _fp_body(x1_ref, xyz2T_ref, f1_ref, f2_ref,
             Wa_ref, ba_ref, Wb_ref, bb_ref,
             *rest, with_cls):
    if with_cls:
        Wc1_ref, bc1_ref, Wc2_ref, bc2_ref, out_ref = rest
    else:
        (out_ref,) = rest
    x1 = x1_ref[0]
    ax = x1[:, 0:1]
    ay = x1[:, 1:2]
    az = x1[:, 2:3]
    bx = xyz2T_ref[0, 0][None, :]
    by = xyz2T_ref[0, 1][None, :]
    bz = xyz2T_ref[0, 2][None, :]
    T = x1.shape[0]
    N2 = xyz2T_ref.shape[2]

    na = (ax * ax + ay * ay) + az * az
    nb = (bx * bx + by * by) + bz * bz
    bmat = jnp.concatenate([bx, by, bz], axis=0)
    dotc = _dot(x1, bmat)
    d = (na + nb) - 2.0 * dotc
    iota = jax.lax.broadcasted_iota(jnp.int32, (T, N2), 1)

    e = d
    ws = []
    ids = []
    for _ in range(3):
        m, idx = _first_min_idx(e, iota, N2)
        dist = jnp.sqrt(jnp.maximum(m, 0.0))
        ws.append(1.0 / (dist + 1e-8))
        ids.append(idx)
        e = jnp.where(iota == idx[:, None], _BIG, e)
    wtot = (ws[0] + ws[1]) + ws[2]
    Wmat = ((ws[0] / wtot)[:, None] * (iota == ids[0][:, None])
            + (ws[1] / wtot)[:, None] * (iota == ids[1][:, None])
            + (ws[2] / wtot)[:, None] * (iota == ids[2][:, None])).astype(jnp.float32)

    interp = _dot(Wmat, f2_ref[0], _EXACT)
    h = jnp.concatenate([interp, f1_ref[0]], axis=1)
    h = jnp.maximum(_dot(h, Wa_ref[...]) + ba_ref[...], 0.0)
    h = jnp.maximum(_dot(h, Wb_ref[...]) + bb_ref[...], 0.0)
    if with_cls:
        h = jnp.maximum(_dot(h, Wc1_ref[...]) + bc1_ref[...], 0.0)
        h = _dot(h, Wc2_ref[...]) + bc2_ref[...]
    out_ref[0] = h


def _fp(xyz1, xyz2T, feats1, feats2, mlp, n_tile, cls=None):
    B, N1, _ = xyz1.shape
    N2 = xyz2T.shape[2]
    C1 = feats1.shape[2]
    C2 = feats2.shape[2]
    (Wa, ba), (Wb, bb) = mlp
    weights = [Wa, ba, Wb, bb]
    Cout = Wb.shape[1]
    if cls is not None:
        (Wc1, bc1), (Wc2, bc2) = cls
        weights += [Wc1, bc1, Wc2, bc2]
        Cout = Wc2.shape[1]
    grid = (B, N1 // n_tile)
    w_specs = [pl.BlockSpec(w.shape, lambda b, s: (0, 0)) for w in weights]
    return pl.pallas_call(
        functools.partial(_fp_body, with_cls=cls is not None),
        grid=grid,
        in_specs=[
            pl.BlockSpec((1, n_tile, 3), lambda b, s: (b, s, 0)),
            pl.BlockSpec((1, 3, N2), lambda b, s: (b, 0, 0)),
            pl.BlockSpec((1, n_tile, C1), lambda b, s: (b, s, 0)),
            pl.BlockSpec((1, N2, C2), lambda b, s: (b, 0, 0)),
        ] + w_specs,
        out_specs=pl.BlockSpec((1, n_tile, Cout), lambda b, s: (b, s, 0)),
        out_shape=jax.ShapeDtypeStruct((B, N1, Cout), jnp.float32),
    )(xyz1, xyz2T, feats1, feats2, *weights)


_NPOINTS = [1024, 256, 64, 16]
_RADIUS = [0.1, 0.2, 0.4, 0.8]
_NSAMPLE = [32, 32, 32, 32]
_SA_TILE = [256, 256, 64, 16]
_FP_TILE = [512, 256, 64, 16]


def kernel(pointcloud, params):
    devs = jax.devices()
    nd = len(devs)
    if nd > 1 and pointcloud.shape[0] % nd == 0:
        mesh = Mesh(np.array(devs), ("d",))
        fwd = shard_map(_forward, mesh=mesh,
                        in_specs=(P("d"), P()), out_specs=P("d"),
                        check_rep=False)
        return fwd(pointcloud, params)
    return _forward(pointcloud, params)


def _forward(pointcloud, params):
    pc = pointcloud.astype(jnp.float32)
    B, N, _ = pc.shape

    def prep(mlp):
        return [(W, b.reshape(1, -1)) for (W, b) in mlp]

    sa_params = [prep(m) for m in params["sa"]]
    fp_params = [prep(m) for m in params["fp"]]
    cls_params = prep(params["cls"])

    xyz = pc[..., 0:3]
    xyzT = jnp.transpose(xyz, (0, 2, 1))
    l_xyz = [xyz]
    l_xyzT = [xyzT]
    l_feats = [pc[..., 3:]]
    pdata = pc
    for k in range(4):
        ctrs = _fps(l_xyzT[k], _NPOINTS[k])
        centers = jnp.transpose(ctrs, (2, 1, 0))
        centersT = jnp.transpose(ctrs, (2, 0, 1))
        nf = _sa(pdata, l_xyzT[k], centers, sa_params[k],
                 _RADIUS[k], _NSAMPLE[k], _SA_TILE[k])
        l_xyz.append(centers)
        l_xyzT.append(centersT)
        l_feats.append(nf)
        pdata = jnp.concatenate([centers, nf], axis=-1)

    f = l_feats[4]
    for lvl in (3, 2, 1):
        f = _fp(l_xyz[lvl], l_xyzT[lvl + 1], l_feats[lvl], f,
                fp_params[lvl], _FP_TILE[lvl])
    out = _fp(l_xyz[0], l_xyzT[1], l_feats[0], f,
              fp_params[0], _FP_TILE[0], cls=cls_params)
    return out
